# baseline (speedup 1.0000x reference)
import functools

import jax
import jax.numpy as jnp
from jax import lax
from jax.experimental import pallas as pl
from jax.experimental.pallas import tpu as pltpu
from jax.experimental.pallas import tpu_sc as plsc

NC, NS = 2, 16
NW = NC * NS
B, S, D = 4096, 50, 128
BPW = B // NW
LANES = 16
DBLK = D // LANES
RPC = 2
NCHUNK = BPW // RPC


NBUF = 8


def _pool_body(x_hbm, table_hbm, out_hbm, idx_v,
               buf0, buf1, buf2, buf3, buf4, buf5, buf6, buf7, out_v,
               sem0, sem1, sem2, sem3, sem4, sem5, sem6, sem7):
    w = lax.axis_index("s") * NC + lax.axis_index("c")
    pltpu.sync_copy(x_hbm.at[w], idx_v)

    bufs = (buf0, buf1, buf2, buf3, buf4, buf5, buf6, buf7)
    sems = (sem0, sem1, sem2, sem3, sem4, sem5, sem6, sem7)

    for p in range(NBUF):
        pltpu.async_copy(table_hbm.at[idx_v.at[p]], bufs[p], sems[p])

    def outer(i, carry):
        for p in range(NBUF):
            c = i * NBUF + p
            buf, sem = bufs[p], sems[p]
            pltpu.make_async_copy(
                table_hbm.at[idx_v.at[c]], buf, sem
            ).wait()

            UNROLL = 5

            def inner(j, accs):
                j0 = j * UNROLL
                out = []
                for r in range(RPC):
                    base = r * S
                    for d in range(DBLK):
                        a = accs[r * DBLK + d]
                        for u in range(UNROLL):
                            a = a + buf[base + j0 + u,
                                        pl.ds(LANES * d, LANES)]
                        out.append(a)
                return tuple(out)

            accs = lax.fori_loop(
                0, S // UNROLL, inner,
                tuple(jnp.zeros((LANES,), jnp.float32)
                      for _ in range(RPC * DBLK)),
            )
            for r in range(RPC):
                for d in range(DBLK):
                    out_v[c * RPC + r, pl.ds(LANES * d, LANES)] = \
                        accs[r * DBLK + d]

            nxt = c + NBUF

            @pl.when(nxt < NCHUNK)
            def _():
                pltpu.async_copy(table_hbm.at[idx_v.at[nxt]], buf, sem)

        return carry

    lax.fori_loop(0, NCHUNK // NBUF, outer, 0)
    pltpu.sync_copy(out_v, out_hbm.at[pl.ds(w * BPW, BPW)])


_pool = pl.kernel(
    _pool_body,
    out_type=jax.ShapeDtypeStruct((B, D), jnp.float32),
    mesh=plsc.VectorSubcoreMesh(core_axis_name="c", subcore_axis_name="s"),
    scratch_types=[
        pltpu.VMEM((NCHUNK, RPC * S), jnp.int32),
        pltpu.VMEM((RPC * S, D), jnp.float32),
        pltpu.VMEM((RPC * S, D), jnp.float32),
        pltpu.VMEM((RPC * S, D), jnp.float32),
        pltpu.VMEM((RPC * S, D), jnp.float32),
        pltpu.VMEM((RPC * S, D), jnp.float32),
        pltpu.VMEM((RPC * S, D), jnp.float32),
        pltpu.VMEM((RPC * S, D), jnp.float32),
        pltpu.VMEM((RPC * S, D), jnp.float32),
        pltpu.VMEM((BPW, D), jnp.float32),
        pltpu.SemaphoreType.DMA,
        pltpu.SemaphoreType.DMA,
        pltpu.SemaphoreType.DMA,
        pltpu.SemaphoreType.DMA,
        pltpu.SemaphoreType.DMA,
        pltpu.SemaphoreType.DMA,
        pltpu.SemaphoreType.DMA,
        pltpu.SemaphoreType.DMA,
    ],
)


def _proj_kernel(s_ref, w_ref, b_ref, o_ref):
    acc = lax.dot_general(
        s_ref[...], w_ref[...],
        (((1,), (1,)), ((), ())),
        preferred_element_type=jnp.float32,
    )
    o_ref[...] = acc * (1.0 / S) + b_ref[...]


def _proj(pooled, W, b2d):
    blk = 2048
    return pl.pallas_call(
        _proj_kernel,
        grid=(B // blk,),
        in_specs=[
            pl.BlockSpec((blk, D), lambda i: (i, 0)),
            pl.BlockSpec((D, D), lambda i: (0, 0)),
            pl.BlockSpec((1, D), lambda i: (0, 0)),
        ],
        out_specs=pl.BlockSpec((blk, D), lambda i: (i, 0)),
        out_shape=jax.ShapeDtypeStruct((B, D), jnp.float32),
    )(pooled, W, b2d)


def kernel(x, table, W, b):
    x32 = x.astype(jnp.int32).reshape(NW, NCHUNK, RPC * S)
    sums = _pool(x32, table)
    return _proj(sums, W, b.reshape(1, D))


# speedup vs baseline: 1.0190x; 1.0190x over previous
import functools

import jax
import jax.numpy as jnp
from jax import lax
from jax.experimental import pallas as pl
from jax.experimental.pallas import tpu as pltpu
from jax.experimental.pallas import tpu_sc as plsc

NC, NS = 2, 16
NW = NC * NS
B, S, D = 4096, 50, 128
BPW = B // NW
LANES = 16
DBLK = D // LANES
RPC = 2
NCHUNK = BPW // RPC


NBUF = 4


def _pool_body(x_hbm, table_hbm, out_hbm, idx_v,
               buf0, buf1, buf2, buf3, out_v,
               sem0, sem1, sem2, sem3):
    w = lax.axis_index("s") * NC + lax.axis_index("c")
    pltpu.sync_copy(x_hbm.at[w], idx_v)

    bufs = (buf0, buf1, buf2, buf3)
    sems = (sem0, sem1, sem2, sem3)

    for p in range(NBUF):
        pltpu.async_copy(table_hbm.at[idx_v.at[p]], bufs[p], sems[p])

    def outer(i, carry):
        for p in range(NBUF):
            c = i * NBUF + p
            buf, sem = bufs[p], sems[p]
            pltpu.make_async_copy(
                table_hbm.at[idx_v.at[c]], buf, sem
            ).wait()

            UNROLL = 5

            def inner(j, accs):
                j0 = j * UNROLL
                out = []
                for r in range(RPC):
                    base = r * S
                    for d in range(DBLK):
                        a = accs[r * DBLK + d]
                        for u in range(UNROLL):
                            a = a + buf[base + j0 + u,
                                        pl.ds(LANES * d, LANES)]
                        out.append(a)
                return tuple(out)

            accs = lax.fori_loop(
                0, S // UNROLL, inner,
                tuple(jnp.zeros((LANES,), jnp.float32)
                      for _ in range(RPC * DBLK)),
            )
            for r in range(RPC):
                for d in range(DBLK):
                    out_v[c * RPC + r, pl.ds(LANES * d, LANES)] = \
                        accs[r * DBLK + d]

            nxt = c + NBUF

            @pl.when(nxt < NCHUNK)
            def _():
                pltpu.async_copy(table_hbm.at[idx_v.at[nxt]], buf, sem)

        return carry

    lax.fori_loop(0, NCHUNK // NBUF, outer, 0)
    pltpu.sync_copy(out_v, out_hbm.at[pl.ds(w * BPW, BPW)])


_pool = pl.kernel(
    _pool_body,
    out_type=jax.ShapeDtypeStruct((B, D), jnp.float32),
    mesh=plsc.VectorSubcoreMesh(core_axis_name="c", subcore_axis_name="s"),
    scratch_types=[
        pltpu.VMEM((NCHUNK, RPC * S), jnp.int32),
        pltpu.VMEM((RPC * S, D), jnp.float32),
        pltpu.VMEM((RPC * S, D), jnp.float32),
        pltpu.VMEM((RPC * S, D), jnp.float32),
        pltpu.VMEM((RPC * S, D), jnp.float32),
        pltpu.VMEM((BPW, D), jnp.float32),
        pltpu.SemaphoreType.DMA,
        pltpu.SemaphoreType.DMA,
        pltpu.SemaphoreType.DMA,
        pltpu.SemaphoreType.DMA,
    ],
)


def _proj_kernel(s_ref, w_ref, b_ref, o_ref):
    acc = lax.dot_general(
        s_ref[...], w_ref[...],
        (((1,), (1,)), ((), ())),
        preferred_element_type=jnp.float32,
    )
    o_ref[...] = acc * (1.0 / S) + b_ref[...]


def _proj(pooled, W, b2d):
    blk = 2048
    return pl.pallas_call(
        _proj_kernel,
        grid=(B // blk,),
        in_specs=[
            pl.BlockSpec((blk, D), lambda i: (i, 0)),
            pl.BlockSpec((D, D), lambda i: (0, 0)),
            pl.BlockSpec((1, D), lambda i: (0, 0)),
        ],
        out_specs=pl.BlockSpec((blk, D), lambda i: (i, 0)),
        out_shape=jax.ShapeDtypeStruct((B, D), jnp.float32),
    )(pooled, W, b2d)


def kernel(x, table, W, b):
    x32 = x.astype(jnp.int32).reshape(NW, NCHUNK, RPC * S)
    sums = _pool(x32, table)
    return _proj(sums, W, b.reshape(1, D))


# speedup vs baseline: 1.0201x; 1.0011x over previous
import functools

import jax
import jax.numpy as jnp
from jax import lax
from jax.experimental import pallas as pl
from jax.experimental.pallas import tpu as pltpu
from jax.experimental.pallas import tpu_sc as plsc

NC, NS = 2, 16
NW = NC * NS
B, S, D = 4096, 50, 128
BPW = B // NW
LANES = 16
DBLK = D // LANES
RPC = 2
NCHUNK = BPW // RPC


NBUF = 4


def _pool_body(x_hbm, table_hbm, out_hbm, idx_v,
               buf0, buf1, buf2, buf3, out_v,
               sem0, sem1, sem2, sem3):
    w = lax.axis_index("s") * NC + lax.axis_index("c")
    pltpu.sync_copy(x_hbm.at[w], idx_v)

    bufs = (buf0, buf1, buf2, buf3)
    sems = (sem0, sem1, sem2, sem3)

    for p in range(NBUF):
        pltpu.async_copy(table_hbm.at[idx_v.at[p]], bufs[p], sems[p])

    def outer(i, carry):
        for p in range(NBUF):
            c = i * NBUF + p
            buf, sem = bufs[p], sems[p]
            pltpu.make_async_copy(
                table_hbm.at[idx_v.at[c]], buf, sem
            ).wait()

            UNROLL = 5

            def inner(j, accs):
                j0 = j * UNROLL
                out = []
                for r in range(RPC):
                    base = r * S
                    for d in range(DBLK):
                        a = accs[r * DBLK + d]
                        for u in range(UNROLL):
                            a = a + buf[base + j0 + u,
                                        pl.ds(LANES * d, LANES)]
                        out.append(a)
                return tuple(out)

            accs = lax.fori_loop(
                0, S // UNROLL, inner,
                tuple(jnp.zeros((LANES,), jnp.float32)
                      for _ in range(RPC * DBLK)),
            )
            for r in range(RPC):
                for d in range(DBLK):
                    out_v[c * RPC + r, pl.ds(LANES * d, LANES)] = \
                        accs[r * DBLK + d]

            nxt = c + NBUF

            @pl.when(nxt < NCHUNK)
            def _():
                pltpu.async_copy(table_hbm.at[idx_v.at[nxt]], buf, sem)

        return carry

    lax.fori_loop(0, NCHUNK // NBUF, outer, 0)
    pltpu.sync_copy(out_v, out_hbm.at[pl.ds(w * BPW, BPW)])


_pool = pl.kernel(
    _pool_body,
    out_type=jax.ShapeDtypeStruct((B, D), jnp.float32),
    mesh=plsc.VectorSubcoreMesh(core_axis_name="c", subcore_axis_name="s"),
    scratch_types=[
        pltpu.VMEM((NCHUNK, RPC * S), jnp.int32),
        pltpu.VMEM((RPC * S, D), jnp.float32),
        pltpu.VMEM((RPC * S, D), jnp.float32),
        pltpu.VMEM((RPC * S, D), jnp.float32),
        pltpu.VMEM((RPC * S, D), jnp.float32),
        pltpu.VMEM((BPW, D), jnp.float32),
        pltpu.SemaphoreType.DMA,
        pltpu.SemaphoreType.DMA,
        pltpu.SemaphoreType.DMA,
        pltpu.SemaphoreType.DMA,
    ],
)


def _proj_kernel(s_ref, w_ref, b_ref, o_ref):
    acc = lax.dot_general(
        s_ref[...], w_ref[...],
        (((1,), (1,)), ((), ())),
        preferred_element_type=jnp.float32,
    )
    o_ref[...] = acc * (1.0 / S) + b_ref[...]


def _proj(pooled, W, b2d):
    blk = 4096
    return pl.pallas_call(
        _proj_kernel,
        grid=(B // blk,),
        in_specs=[
            pl.BlockSpec((blk, D), lambda i: (i, 0)),
            pl.BlockSpec((D, D), lambda i: (0, 0)),
            pl.BlockSpec((1, D), lambda i: (0, 0)),
        ],
        out_specs=pl.BlockSpec((blk, D), lambda i: (i, 0)),
        out_shape=jax.ShapeDtypeStruct((B, D), jnp.float32),
    )(pooled, W, b2d)


def kernel(x, table, W, b):
    x32 = x.astype(jnp.int32).reshape(NW, NCHUNK, RPC * S)
    sums = _pool(x32, table)
    return _proj(sums, W, b.reshape(1, D))


# speedup vs baseline: 1.0212x; 1.0011x over previous
import functools

import jax
import jax.numpy as jnp
from jax import lax
from jax.experimental import pallas as pl
from jax.experimental.pallas import tpu as pltpu
from jax.experimental.pallas import tpu_sc as plsc

NC, NS = 2, 16
NW = NC * NS
B, S, D = 4096, 50, 128
BPW = B // NW
LANES = 16
DBLK = D // LANES
RPC = 2
NCHUNK = BPW // RPC


NBUF = 4


def _pool_body(x_hbm, table_hbm, out_hbm, idx_v,
               buf0, buf1, buf2, buf3, out_v,
               sem0, sem1, sem2, sem3):
    w = lax.axis_index("s") * NC + lax.axis_index("c")
    pltpu.sync_copy(x_hbm.at[w], idx_v)

    bufs = (buf0, buf1, buf2, buf3)
    sems = (sem0, sem1, sem2, sem3)

    for p in range(NBUF):
        pltpu.async_copy(table_hbm.at[idx_v.at[p]], bufs[p], sems[p])

    def outer(i, carry):
        for p in range(NBUF):
            c = i * NBUF + p
            buf, sem = bufs[p], sems[p]
            pltpu.make_async_copy(
                table_hbm.at[idx_v.at[c]], buf, sem
            ).wait()

            UNROLL = 5

            def inner(j, accs):
                j0 = j * UNROLL
                out = []
                for r in range(RPC):
                    base = r * S
                    for d in range(DBLK):
                        a = accs[r * DBLK + d]
                        for u in range(UNROLL):
                            a = a + buf[base + j0 + u,
                                        pl.ds(LANES * d, LANES)]
                        out.append(a)
                return tuple(out)

            accs = lax.fori_loop(
                0, S // UNROLL, inner,
                tuple(jnp.zeros((LANES,), jnp.float32)
                      for _ in range(RPC * DBLK)),
            )
            for r in range(RPC):
                for d in range(DBLK):
                    out_v[c * RPC + r, pl.ds(LANES * d, LANES)] = \
                        accs[r * DBLK + d]

            nxt = c + NBUF

            @pl.when(nxt < NCHUNK)
            def _():
                pltpu.async_copy(table_hbm.at[idx_v.at[nxt]], buf, sem)

        return carry

    lax.fori_loop(0, NCHUNK // NBUF, outer, 0)
    pltpu.sync_copy(out_v, out_hbm.at[pl.ds(w * BPW, BPW)])


_pool = pl.kernel(
    _pool_body,
    out_type=jax.ShapeDtypeStruct((B, D), jnp.float32),
    mesh=plsc.VectorSubcoreMesh(core_axis_name="c", subcore_axis_name="s"),
    scratch_types=[
        pltpu.VMEM((NCHUNK, RPC * S), jnp.int32),
        pltpu.VMEM((RPC * S, D), jnp.float32),
        pltpu.VMEM((RPC * S, D), jnp.float32),
        pltpu.VMEM((RPC * S, D), jnp.float32),
        pltpu.VMEM((RPC * S, D), jnp.float32),
        pltpu.VMEM((BPW, D), jnp.float32),
        pltpu.SemaphoreType.DMA,
        pltpu.SemaphoreType.DMA,
        pltpu.SemaphoreType.DMA,
        pltpu.SemaphoreType.DMA,
    ],
)


def _proj_kernel(s_ref, w_ref, b_ref, o_ref):
    acc = lax.dot_general(
        s_ref[...], w_ref[...],
        (((1,), (1,)), ((), ())),
        preferred_element_type=jnp.float32,
    )
    o_ref[...] = acc * (1.0 / S) + b_ref[...]


def _proj(pooled, W, b2d):
    blk = 2048
    return pl.pallas_call(
        _proj_kernel,
        grid=(B // blk,),
        in_specs=[
            pl.BlockSpec((blk, D), lambda i: (i, 0)),
            pl.BlockSpec((D, D), lambda i: (0, 0)),
            pl.BlockSpec((1, D), lambda i: (0, 0)),
        ],
        out_specs=pl.BlockSpec((blk, D), lambda i: (i, 0)),
        out_shape=jax.ShapeDtypeStruct((B, D), jnp.float32),
    )(pooled, W, b2d)


def kernel(x, table, W, b):
    x32 = x.astype(jnp.int32).reshape(NW, NCHUNK, RPC * S)
    sums = _pool(x32, table)
    return _proj(sums, W, b.reshape(1, D))


# speedup vs baseline: 1.0378x; 1.0162x over previous
import functools

import jax
import jax.numpy as jnp
from jax import lax
from jax.experimental import pallas as pl
from jax.experimental.pallas import tpu as pltpu
from jax.experimental.pallas import tpu_sc as plsc

NC, NS = 2, 16
NW = NC * NS
B, S, D = 4096, 50, 128
BPW = B // NW
LANES = 16
DBLK = D // LANES
RPC = 1
NCHUNK = BPW // RPC


NBUF = 8


def _pool_body(x_hbm, table_hbm, out_hbm, idx_v,
               buf0, buf1, buf2, buf3, buf4, buf5, buf6, buf7, out_v,
               sem0, sem1, sem2, sem3, sem4, sem5, sem6, sem7):
    w = lax.axis_index("s") * NC + lax.axis_index("c")
    pltpu.sync_copy(x_hbm.at[pl.ds(w * BPW, BPW)], idx_v)

    bufs = (buf0, buf1, buf2, buf3, buf4, buf5, buf6, buf7)
    sems = (sem0, sem1, sem2, sem3, sem4, sem5, sem6, sem7)

    for p in range(NBUF):
        pltpu.async_copy(table_hbm.at[idx_v.at[p]], bufs[p], sems[p])

    def outer(i, carry):
        for p in range(NBUF):
            c = i * NBUF + p
            buf, sem = bufs[p], sems[p]
            pltpu.make_async_copy(
                table_hbm.at[idx_v.at[c]], buf, sem
            ).wait()

            UNROLL = 5

            def inner(j, accs):
                j0 = j * UNROLL
                out = []
                for r in range(RPC):
                    base = r * S
                    for d in range(DBLK):
                        a = accs[r * DBLK + d]
                        for u in range(UNROLL):
                            a = a + buf[base + j0 + u,
                                        pl.ds(LANES * d, LANES)]
                        out.append(a)
                return tuple(out)

            accs = lax.fori_loop(
                0, S // UNROLL, inner,
                tuple(jnp.zeros((LANES,), jnp.float32)
                      for _ in range(RPC * DBLK)),
            )
            for r in range(RPC):
                for d in range(DBLK):
                    out_v[c * RPC + r, pl.ds(LANES * d, LANES)] = \
                        accs[r * DBLK + d]

            nxt = c + NBUF

            @pl.when(nxt < NCHUNK)
            def _():
                pltpu.async_copy(table_hbm.at[idx_v.at[nxt]], buf, sem)

        return carry

    lax.fori_loop(0, NCHUNK // NBUF, outer, 0)
    pltpu.sync_copy(out_v, out_hbm.at[pl.ds(w * BPW, BPW)])


_pool = pl.kernel(
    _pool_body,
    out_type=jax.ShapeDtypeStruct((B, D), jnp.float32),
    mesh=plsc.VectorSubcoreMesh(core_axis_name="c", subcore_axis_name="s"),
    scratch_types=[
        pltpu.VMEM((BPW, S), jnp.int32),
        pltpu.VMEM((RPC * S, D), jnp.float32),
        pltpu.VMEM((RPC * S, D), jnp.float32),
        pltpu.VMEM((RPC * S, D), jnp.float32),
        pltpu.VMEM((RPC * S, D), jnp.float32),
        pltpu.VMEM((RPC * S, D), jnp.float32),
        pltpu.VMEM((RPC * S, D), jnp.float32),
        pltpu.VMEM((RPC * S, D), jnp.float32),
        pltpu.VMEM((RPC * S, D), jnp.float32),
        pltpu.VMEM((BPW, D), jnp.float32),
        pltpu.SemaphoreType.DMA,
        pltpu.SemaphoreType.DMA,
        pltpu.SemaphoreType.DMA,
        pltpu.SemaphoreType.DMA,
        pltpu.SemaphoreType.DMA,
        pltpu.SemaphoreType.DMA,
        pltpu.SemaphoreType.DMA,
        pltpu.SemaphoreType.DMA,
    ],
)


def _proj_kernel(s_ref, w_ref, b_ref, o_ref):
    acc = lax.dot_general(
        s_ref[...], w_ref[...],
        (((1,), (1,)), ((), ())),
        preferred_element_type=jnp.float32,
    )
    o_ref[...] = acc * (1.0 / S) + b_ref[...]


def _proj(pooled, W, b2d):
    blk = 2048
    return pl.pallas_call(
        _proj_kernel,
        grid=(B // blk,),
        in_specs=[
            pl.BlockSpec((blk, D), lambda i: (i, 0)),
            pl.BlockSpec((D, D), lambda i: (0, 0)),
            pl.BlockSpec((1, D), lambda i: (0, 0)),
        ],
        out_specs=pl.BlockSpec((blk, D), lambda i: (i, 0)),
        out_shape=jax.ShapeDtypeStruct((B, D), jnp.float32),
    )(pooled, W, b2d)


def kernel(x, table, W, b):
    sums = _pool(x.astype(jnp.int32), table)
    return _proj(sums, W, b.reshape(1, D))
